# baseline (speedup 1.0000x reference)
import functools

import jax
import jax.numpy as jnp
from jax import lax
from jax.experimental import pallas as pl
from jax.experimental.pallas import tpu as pltpu
from jax.experimental.pallas import tpu_sc as plsc

S = 2048
D = 1024
H = 16
HD = 64
E = 8
F = 4096
T = S
TILE = 128
NT = 40
PADT = NT * TILE
NEG = -1e30

_NC = 2
_NW = _NC * 16
_TPW = T // _NW


def _qkv_body(x_ref, w_ref, b_ref, o_ref):
    o_ref[...] = (
        jnp.dot(x_ref[...], w_ref[...], preferred_element_type=jnp.float32)
        + b_ref[0:1, :]
    )


def _attn_body(q_ref, k_ref, v_ref, o_ref):
    qi = pl.program_id(1)
    q = q_ref[0] * (1.0 / 8.0)
    s = lax.dot_general(q, k_ref[0], (((1,), (1,)), ((), ())),
                        preferred_element_type=jnp.float32)
    row = qi * 128 + lax.broadcasted_iota(jnp.int32, (128, S), 0)
    col = lax.broadcasted_iota(jnp.int32, (128, S), 1)
    s = jnp.where(col <= row, s, NEG)
    m = jnp.max(s, axis=-1, keepdims=True)
    p = jnp.exp(s - m)
    p = p / jnp.sum(p, axis=-1, keepdims=True)
    o_ref[0] = jnp.dot(p, v_ref[0], preferred_element_type=jnp.float32)


def _proj_body(ctx_ref, w_ref, b_ref, x_ref, o_ref):
    h = pl.program_id(1)

    @pl.when(h == 0)
    def _():
        o_ref[...] = x_ref[...] + b_ref[0:1, :]

    o_ref[...] += jnp.dot(ctx_ref[0], w_ref[0],
                          preferred_element_type=jnp.float32)


def _route_body(h_ref, rw_ref, rb_ref, cw_ref, dest_ref, te_ref):
    hdn = h_ref[...]
    logits = (jnp.dot(hdn, rw_ref[...], preferred_element_type=jnp.float32)
              + rb_ref[0:1, :])
    lane = lax.broadcasted_iota(jnp.int32, (T, 128), 1)
    logits = jnp.where(lane < E, logits, NEG)
    m = jnp.max(logits, axis=-1, keepdims=True)
    p = jnp.exp(logits - m)
    probs = p / jnp.sum(p, axis=-1, keepdims=True)
    m1 = jnp.max(probs, axis=-1, keepdims=True)
    i1 = jnp.min(jnp.where(probs == m1, lane, 128), axis=-1, keepdims=True)
    probs2 = jnp.where(lane == i1, -1.0, probs)
    m2 = jnp.max(probs2, axis=-1, keepdims=True)
    i2 = jnp.min(jnp.where(probs2 == m2, lane, 128), axis=-1, keepdims=True)
    den = m1 + m2
    cw1 = m1 / den
    cw2 = m2 / den
    am = ((lane == i1) | (lane == i2)).astype(jnp.int32)
    cs = am
    sh = 1
    while sh < T:
        z = jnp.zeros((sh, 128), jnp.int32)
        cs = cs + jnp.concatenate([z, cs[:-sh, :]], axis=0)
        sh *= 2
    ranks = cs - am
    counts = cs[T - 1:T, :]
    tiles_e = (counts + (TILE - 1)) // TILE
    tef = jnp.broadcast_to(tiles_e.astype(jnp.float32), (8, 128))
    r_i = lax.broadcasted_iota(jnp.int32, (128, 128), 0)
    c_i = lax.broadcasted_iota(jnp.int32, (128, 128), 1)
    tri = (r_i <= c_i).astype(jnp.float32)
    cum_i = jnp.dot(tef, tri,
                    preferred_element_type=jnp.float32)[0:1, :].astype(jnp.int32)
    po = (cum_i - tiles_e) * TILE
    slot = po + ranks
    d1 = jnp.sum(jnp.where(lane == i1, slot, 0), axis=-1, keepdims=True)
    d2 = jnp.sum(jnp.where(lane == i2, slot, 0), axis=-1, keepdims=True)
    cw_ref[...] = jnp.where(lane == 0, cw1, jnp.where(lane == 1, cw2, 0.0))
    dest_ref[...] = jnp.where(lane == 0, d1, jnp.where(lane == 1, d2, 0))
    eye = (lax.broadcasted_iota(jnp.int32, (8, 128), 0)
           == lax.broadcasted_iota(jnp.int32, (8, 128), 1))
    cum_col = jnp.sum(jnp.where(eye, jnp.broadcast_to(cum_i, (8, 128)), 0),
                      axis=-1, keepdims=True)
    lane8 = lax.broadcasted_iota(jnp.int32, (8, 128), 1)
    te2d = jnp.sum((lane8 >= cum_col).astype(jnp.int32), axis=0, keepdims=True)
    te_ref[...] = jnp.broadcast_to(jnp.minimum(te2d, E - 1), (8, 128))


def _gmm1_body(te_sref, x_ref, w1_ref, b1_ref, o_ref):
    h1 = (jnp.dot(x_ref[...], w1_ref[0], preferred_element_type=jnp.float32)
          + b1_ref[0])
    o_ref[...] = jax.nn.gelu(h1, approximate=True)


def _gmm2_body(te_sref, h1_ref, w2_ref, b2_ref, o_ref):
    o_ref[...] = (jnp.dot(h1_ref[...], w2_ref[0],
                          preferred_element_type=jnp.float32) + b2_ref[0])


def _fin_body(h_ref, g0_ref, g1_ref, cw_ref, o_ref):
    o_ref[...] = (h_ref[...] + cw_ref[:, 0:1] * g0_ref[...]
                  + cw_ref[:, 1:2] * g1_ref[...])


def _sc_mesh():
    return plsc.VectorSubcoreMesh(core_axis_name="c", subcore_axis_name="s")


def _sc_dispatch(hidden, d0, d1):
    @functools.partial(
        pl.kernel, mesh=_sc_mesh(),
        out_type=jax.ShapeDtypeStruct((PADT, D), jnp.float32),
        scratch_types=[pltpu.VMEM((_TPW, D), jnp.float32),
                       pltpu.VMEM((_TPW,), jnp.int32),
                       pltpu.SemaphoreType.DMA],
    )
    def run(h_hbm, d0_hbm, d1_hbm, x_hbm, h_v, idx_v, sem):
        wid = lax.axis_index("s") * _NC + lax.axis_index("c")
        base = wid * _TPW
        pltpu.sync_copy(h_hbm.at[pl.ds(base, _TPW)], h_v)
        pltpu.sync_copy(d0_hbm.at[pl.ds(base, _TPW)], idx_v)
        pltpu.async_copy(h_v, x_hbm.at[idx_v], sem).wait()
        pltpu.sync_copy(d1_hbm.at[pl.ds(base, _TPW)], idx_v)
        pltpu.async_copy(h_v, x_hbm.at[idx_v], sem).wait()

    return run(hidden, d0, d1)


def _sc_combine(y, idx_all):
    @functools.partial(
        pl.kernel, mesh=_sc_mesh(),
        out_type=jax.ShapeDtypeStruct((2 * T, D), jnp.float32),
        scratch_types=[pltpu.VMEM((_TPW, D), jnp.float32),
                       pltpu.VMEM((_TPW,), jnp.int32),
                       pltpu.SemaphoreType.DMA],
    )
    def run(y_hbm, idx_hbm, g_hbm, rows_v, idx_v, sem):
        wid = lax.axis_index("s") * _NC + lax.axis_index("c")
        for cch in range(2):
            base = wid * (2 * _TPW) + cch * _TPW
            pltpu.sync_copy(idx_hbm.at[pl.ds(base, _TPW)], idx_v)
            pltpu.async_copy(y_hbm.at[idx_v], rows_v, sem).wait()
            pltpu.sync_copy(rows_v, g_hbm.at[pl.ds(base, _TPW)])

    return run(y, idx_all)


def kernel(hidden_states, c_attn_w, c_attn_b, c_proj_w, c_proj_b,
           router_w, router_b, expert_w1, expert_b1, expert_w2, expert_b2):
    x = hidden_states.reshape(S, D)

    b2d = jnp.broadcast_to(c_attn_b.reshape(1, 3 * D), (8, 3 * D))
    qkv = pl.pallas_call(
        _qkv_body,
        grid=(6, 8),
        in_specs=[pl.BlockSpec((256, D), lambda j, i: (i, 0)),
                  pl.BlockSpec((D, 512), lambda j, i: (0, j)),
                  pl.BlockSpec((8, 512), lambda j, i: (0, j))],
        out_specs=pl.BlockSpec((256, 512), lambda j, i: (i, j)),
        out_shape=jax.ShapeDtypeStruct((S, 3 * D), jnp.float32),
    )(x, c_attn_w, b2d)

    q = qkv[:, 0:D].reshape(S, H, HD).transpose(1, 0, 2)
    k = qkv[:, D:2 * D].reshape(S, H, HD).transpose(1, 0, 2)
    v = qkv[:, 2 * D:].reshape(S, H, HD).transpose(1, 0, 2)
    ctx = pl.pallas_call(
        _attn_body,
        grid=(H, S // 128),
        in_specs=[pl.BlockSpec((1, 128, HD), lambda h, i: (h, i, 0)),
                  pl.BlockSpec((1, S, HD), lambda h, i: (h, 0, 0)),
                  pl.BlockSpec((1, S, HD), lambda h, i: (h, 0, 0))],
        out_specs=pl.BlockSpec((1, 128, HD), lambda h, i: (h, i, 0)),
        out_shape=jax.ShapeDtypeStruct((H, S, HD), jnp.float32),
        compiler_params=pltpu.CompilerParams(
            vmem_limit_bytes=100 * 1024 * 1024),
    )(q, k, v)

    wp = c_proj_w.reshape(H, HD, D)
    bp = jnp.broadcast_to(c_proj_b.reshape(1, D), (8, D))
    hidden = pl.pallas_call(
        _proj_body,
        grid=(S // 128, H),
        in_specs=[pl.BlockSpec((1, 128, HD), lambda i, h: (h, i, 0)),
                  pl.BlockSpec((1, HD, D), lambda i, h: (h, 0, 0)),
                  pl.BlockSpec((8, D), lambda i, h: (0, 0)),
                  pl.BlockSpec((128, D), lambda i, h: (i, 0))],
        out_specs=pl.BlockSpec((128, D), lambda i, h: (i, 0)),
        out_shape=jax.ShapeDtypeStruct((S, D), jnp.float32),
        compiler_params=pltpu.CompilerParams(
            dimension_semantics=("arbitrary", "arbitrary")),
    )(ctx, wp, bp, x)

    rw = jnp.pad(router_w, ((0, 0), (0, 128 - E)))
    rb = jnp.pad(router_b.reshape(1, E), ((0, 7), (0, 128 - E)))
    cwA, destB, teC = pl.pallas_call(
        _route_body,
        grid=(1,),
        in_specs=[pl.BlockSpec((T, D), lambda i: (0, 0)),
                  pl.BlockSpec((D, 128), lambda i: (0, 0)),
                  pl.BlockSpec((8, 128), lambda i: (0, 0))],
        out_specs=[pl.BlockSpec((T, 128), lambda i: (0, 0)),
                   pl.BlockSpec((T, 128), lambda i: (0, 0)),
                   pl.BlockSpec((8, 128), lambda i: (0, 0))],
        out_shape=[jax.ShapeDtypeStruct((T, 128), jnp.float32),
                   jax.ShapeDtypeStruct((T, 128), jnp.int32),
                   jax.ShapeDtypeStruct((8, 128), jnp.int32)],
        compiler_params=pltpu.CompilerParams(
            vmem_limit_bytes=100 * 1024 * 1024),
    )(hidden, rw, rb)
    d0 = destB[:, 0]
    d1 = destB[:, 1]
    te = teC[0, :NT]

    xs = _sc_dispatch(hidden, d0, d1)

    b1r = expert_b1.reshape(E, 1, F)
    b2r = expert_b2.reshape(E, 1, D)
    h1 = pl.pallas_call(
        _gmm1_body,
        grid_spec=pltpu.PrefetchScalarGridSpec(
            num_scalar_prefetch=1,
            grid=(NT,),
            in_specs=[pl.BlockSpec((TILE, D), lambda i, te_s: (i, 0)),
                      pl.BlockSpec((1, D, F), lambda i, te_s: (te_s[i], 0, 0)),
                      pl.BlockSpec((1, 1, F), lambda i, te_s: (te_s[i], 0, 0))],
            out_specs=pl.BlockSpec((TILE, F), lambda i, te_s: (i, 0)),
        ),
        out_shape=jax.ShapeDtypeStruct((PADT, F), jnp.float32),
        compiler_params=pltpu.CompilerParams(
            vmem_limit_bytes=100 * 1024 * 1024),
    )(te, xs, expert_w1, b1r)
    y = pl.pallas_call(
        _gmm2_body,
        grid_spec=pltpu.PrefetchScalarGridSpec(
            num_scalar_prefetch=1,
            grid=(NT,),
            in_specs=[pl.BlockSpec((TILE, F), lambda i, te_s: (i, 0)),
                      pl.BlockSpec((1, F, D), lambda i, te_s: (te_s[i], 0, 0)),
                      pl.BlockSpec((1, 1, D), lambda i, te_s: (te_s[i], 0, 0))],
            out_specs=pl.BlockSpec((TILE, D), lambda i, te_s: (i, 0)),
        ),
        out_shape=jax.ShapeDtypeStruct((PADT, D), jnp.float32),
        compiler_params=pltpu.CompilerParams(
            vmem_limit_bytes=100 * 1024 * 1024),
    )(te, h1, expert_w2, b2r)

    idx_all = jnp.concatenate([d0, d1], axis=0)
    g = _sc_combine(y, idx_all)

    out = pl.pallas_call(
        _fin_body,
        grid=(S // 128,),
        in_specs=[pl.BlockSpec((128, D), lambda i: (i, 0)),
                  pl.BlockSpec((128, D), lambda i: (i, 0)),
                  pl.BlockSpec((128, D), lambda i: (i + S // 128, 0)),
                  pl.BlockSpec((128, 128), lambda i: (i, 0))],
        out_specs=pl.BlockSpec((128, D), lambda i: (i, 0)),
        out_shape=jax.ShapeDtypeStruct((S, D), jnp.float32),
    )(hidden, g, g, cwA)
    return out.reshape(1, S, D)


# speedup vs baseline: 1.3386x; 1.3386x over previous
import functools

import jax
import jax.numpy as jnp
from jax import lax
from jax.experimental import pallas as pl
from jax.experimental.pallas import tpu as pltpu
from jax.experimental.pallas import tpu_sc as plsc

S = 2048
D = 1024
H = 16
HD = 64
E = 8
F = 4096
T = S
TILE = 128
NT = 40
PADT = NT * TILE
NEG = -1e30

_NC = 2
_NW = _NC * 16
_TPW = T // _NW


def _bdot(a, b):
    return jnp.dot(a.astype(jnp.bfloat16), b.astype(jnp.bfloat16),
                   preferred_element_type=jnp.float32)

def _qkv_body(x_ref, w_ref, b_ref, o_ref):
    o_ref[...] = (
        _bdot(x_ref[...], w_ref[...]) + b_ref[0:1, :]
    )


def _attn_body(q_ref, k_ref, v_ref, o_ref):
    qi = pl.program_id(1)
    q = q_ref[0] * (1.0 / 8.0)
    s = lax.dot_general(q.astype(jnp.bfloat16), k_ref[0].astype(jnp.bfloat16),
                        (((1,), (1,)), ((), ())),
                        preferred_element_type=jnp.float32)
    row = qi * 256 + lax.broadcasted_iota(jnp.int32, (256, S), 0)
    col = lax.broadcasted_iota(jnp.int32, (256, S), 1)
    s = jnp.where(col <= row, s, NEG)
    m = jnp.max(s, axis=-1, keepdims=True)
    p = jnp.exp(s - m)
    l = jnp.sum(p, axis=-1, keepdims=True)
    o_ref[0] = _bdot(p, v_ref[0]) * (1.0 / l)


def _proj_body(c_ref, w_ref, b_ref, x_ref, o_ref):
    o_ref[...] = x_ref[...] + b_ref[0:1, :] + _bdot(c_ref[...], w_ref[...])


def _route_body(h_ref, rw_ref, rb_ref, cw_ref, dest_ref, te_ref):
    hdn = h_ref[...]
    logits = _bdot(hdn, rw_ref[...]) + rb_ref[0:1, :]
    lane = lax.broadcasted_iota(jnp.int32, (T, 128), 1)
    logits = jnp.where(lane < E, logits, NEG)
    m = jnp.max(logits, axis=-1, keepdims=True)
    p = jnp.exp(logits - m)
    probs = p / jnp.sum(p, axis=-1, keepdims=True)
    m1 = jnp.max(probs, axis=-1, keepdims=True)
    i1 = jnp.min(jnp.where(probs == m1, lane, 128), axis=-1, keepdims=True)
    probs2 = jnp.where(lane == i1, -1.0, probs)
    m2 = jnp.max(probs2, axis=-1, keepdims=True)
    i2 = jnp.min(jnp.where(probs2 == m2, lane, 128), axis=-1, keepdims=True)
    den = m1 + m2
    cw1 = m1 / den
    cw2 = m2 / den
    am = ((lane == i1) | (lane == i2)).astype(jnp.int32)
    cs = am
    sh = 1
    while sh < T:
        z = jnp.zeros((sh, 128), jnp.int32)
        cs = cs + jnp.concatenate([z, cs[:-sh, :]], axis=0)
        sh *= 2
    ranks = cs - am
    counts = cs[T - 1:T, :]
    tiles_e = (counts + (TILE - 1)) // TILE
    tef = jnp.broadcast_to(tiles_e.astype(jnp.float32), (8, 128))
    r_i = lax.broadcasted_iota(jnp.int32, (128, 128), 0)
    c_i = lax.broadcasted_iota(jnp.int32, (128, 128), 1)
    tri = (r_i <= c_i).astype(jnp.float32)
    cum_i = jnp.dot(tef, tri,
                    preferred_element_type=jnp.float32)[0:1, :].astype(jnp.int32)
    po = (cum_i - tiles_e) * TILE
    slot = po + ranks
    d1 = jnp.sum(jnp.where(lane == i1, slot, 0), axis=-1, keepdims=True)
    d2 = jnp.sum(jnp.where(lane == i2, slot, 0), axis=-1, keepdims=True)
    cw_ref[...] = jnp.where(lane == 0, cw1, jnp.where(lane == 1, cw2, 0.0))
    dest_ref[...] = jnp.where(lane == 0, d1, jnp.where(lane == 1, d2, 0))
    eye = (lax.broadcasted_iota(jnp.int32, (8, 128), 0)
           == lax.broadcasted_iota(jnp.int32, (8, 128), 1))
    cum_col = jnp.sum(jnp.where(eye, jnp.broadcast_to(cum_i, (8, 128)), 0),
                      axis=-1, keepdims=True)
    lane8 = lax.broadcasted_iota(jnp.int32, (8, 128), 1)
    te2d = jnp.sum((lane8 >= cum_col).astype(jnp.int32), axis=0, keepdims=True)
    te_ref[...] = jnp.broadcast_to(jnp.minimum(te2d, E - 1), (8, 128))


def _gmm1_body(te_sref, x_ref, w1_ref, b1_ref, o_ref):
    h1 = _bdot(x_ref[...], w1_ref[0]) + b1_ref[0]
    o_ref[...] = jax.nn.gelu(h1, approximate=True)


def _gmm2_body(te_sref, h1_ref, w2_ref, b2_ref, o_ref):
    o_ref[...] = _bdot(h1_ref[...], w2_ref[0]) + b2_ref[0]


def _fin_body(h_ref, g0_ref, g1_ref, cw_ref, o_ref):
    o_ref[...] = (h_ref[...] + cw_ref[:, 0:1] * g0_ref[...]
                  + cw_ref[:, 1:2] * g1_ref[...])


def _sc_mesh():
    return plsc.VectorSubcoreMesh(core_axis_name="c", subcore_axis_name="s")


def _sc_dispatch(hidden, d0, d1):
    @functools.partial(
        pl.kernel, mesh=_sc_mesh(),
        out_type=jax.ShapeDtypeStruct((PADT, D), jnp.float32),
        scratch_types=[pltpu.VMEM((_TPW, D), jnp.float32),
                       pltpu.VMEM((_TPW,), jnp.int32),
                       pltpu.SemaphoreType.DMA],
    )
    def run(h_hbm, d0_hbm, d1_hbm, x_hbm, h_v, idx_v, sem):
        wid = lax.axis_index("s") * _NC + lax.axis_index("c")
        base = wid * _TPW
        pltpu.sync_copy(h_hbm.at[pl.ds(base, _TPW)], h_v)
        pltpu.sync_copy(d0_hbm.at[pl.ds(base, _TPW)], idx_v)
        pltpu.async_copy(h_v, x_hbm.at[idx_v], sem).wait()
        pltpu.sync_copy(d1_hbm.at[pl.ds(base, _TPW)], idx_v)
        pltpu.async_copy(h_v, x_hbm.at[idx_v], sem).wait()

    return run(hidden, d0, d1)


def _sc_combine(y, idx_all):
    @functools.partial(
        pl.kernel, mesh=_sc_mesh(),
        out_type=jax.ShapeDtypeStruct((2 * T, D), jnp.float32),
        scratch_types=[pltpu.VMEM((_TPW, D), jnp.float32),
                       pltpu.VMEM((_TPW,), jnp.int32),
                       pltpu.SemaphoreType.DMA],
    )
    def run(y_hbm, idx_hbm, g_hbm, rows_v, idx_v, sem):
        wid = lax.axis_index("s") * _NC + lax.axis_index("c")
        for cch in range(2):
            base = wid * (2 * _TPW) + cch * _TPW
            pltpu.sync_copy(idx_hbm.at[pl.ds(base, _TPW)], idx_v)
            pltpu.async_copy(y_hbm.at[idx_v], rows_v, sem).wait()
            pltpu.sync_copy(rows_v, g_hbm.at[pl.ds(base, _TPW)])

    return run(y, idx_all)


def kernel(hidden_states, c_attn_w, c_attn_b, c_proj_w, c_proj_b,
           router_w, router_b, expert_w1, expert_b1, expert_w2, expert_b2):
    x = hidden_states.reshape(S, D)

    b2d = jnp.broadcast_to(c_attn_b.reshape(1, 3 * D), (8, 3 * D))
    qkv = pl.pallas_call(
        _qkv_body,
        grid=(6, 8),
        in_specs=[pl.BlockSpec((256, D), lambda j, i: (i, 0)),
                  pl.BlockSpec((D, 512), lambda j, i: (0, j)),
                  pl.BlockSpec((8, 512), lambda j, i: (0, j))],
        out_specs=pl.BlockSpec((256, 512), lambda j, i: (i, j)),
        out_shape=jax.ShapeDtypeStruct((S, 3 * D), jnp.float32),
    )(x, c_attn_w, b2d)

    q = qkv[:, 0:D].reshape(S, H, HD).transpose(1, 0, 2)
    k = qkv[:, D:2 * D].reshape(S, H, HD).transpose(1, 0, 2)
    v = qkv[:, 2 * D:].reshape(S, H, HD).transpose(1, 0, 2)
    ctx = pl.pallas_call(
        _attn_body,
        grid=(H, S // 256),
        in_specs=[pl.BlockSpec((1, 256, HD), lambda h, i: (h, i, 0)),
                  pl.BlockSpec((1, S, HD), lambda h, i: (h, 0, 0)),
                  pl.BlockSpec((1, S, HD), lambda h, i: (h, 0, 0))],
        out_specs=pl.BlockSpec((1, 256, HD), lambda h, i: (h, i, 0)),
        out_shape=jax.ShapeDtypeStruct((H, S, HD), jnp.float32),
        compiler_params=pltpu.CompilerParams(
            vmem_limit_bytes=100 * 1024 * 1024),
    )(q, k, v)

    ctx_flat = ctx.transpose(1, 0, 2).reshape(S, D)
    bp = jnp.broadcast_to(c_proj_b.reshape(1, D), (8, D))
    hidden = pl.pallas_call(
        _proj_body,
        grid=(2, 8),
        in_specs=[pl.BlockSpec((256, D), lambda j, i: (i, 0)),
                  pl.BlockSpec((D, 512), lambda j, i: (0, j)),
                  pl.BlockSpec((8, 512), lambda j, i: (0, j)),
                  pl.BlockSpec((256, 512), lambda j, i: (i, j))],
        out_specs=pl.BlockSpec((256, 512), lambda j, i: (i, j)),
        out_shape=jax.ShapeDtypeStruct((S, D), jnp.float32),
    )(ctx_flat, c_proj_w, bp, x)

    rw = jnp.pad(router_w, ((0, 0), (0, 128 - E)))
    rb = jnp.pad(router_b.reshape(1, E), ((0, 7), (0, 128 - E)))
    cwA, destB, teC = pl.pallas_call(
        _route_body,
        grid=(1,),
        in_specs=[pl.BlockSpec((T, D), lambda i: (0, 0)),
                  pl.BlockSpec((D, 128), lambda i: (0, 0)),
                  pl.BlockSpec((8, 128), lambda i: (0, 0))],
        out_specs=[pl.BlockSpec((T, 128), lambda i: (0, 0)),
                   pl.BlockSpec((T, 128), lambda i: (0, 0)),
                   pl.BlockSpec((8, 128), lambda i: (0, 0))],
        out_shape=[jax.ShapeDtypeStruct((T, 128), jnp.float32),
                   jax.ShapeDtypeStruct((T, 128), jnp.int32),
                   jax.ShapeDtypeStruct((8, 128), jnp.int32)],
        compiler_params=pltpu.CompilerParams(
            vmem_limit_bytes=100 * 1024 * 1024),
    )(hidden, rw, rb)
    d0 = destB[:, 0]
    d1 = destB[:, 1]
    te = teC[0, :NT]

    xs = _sc_dispatch(hidden, d0, d1)

    b1r = expert_b1.reshape(E, 1, F)
    b2r = expert_b2.reshape(E, 1, D)
    h1 = pl.pallas_call(
        _gmm1_body,
        grid_spec=pltpu.PrefetchScalarGridSpec(
            num_scalar_prefetch=1,
            grid=(NT,),
            in_specs=[pl.BlockSpec((TILE, D), lambda i, te_s: (i, 0)),
                      pl.BlockSpec((1, D, F), lambda i, te_s: (te_s[i], 0, 0)),
                      pl.BlockSpec((1, 1, F), lambda i, te_s: (te_s[i], 0, 0))],
            out_specs=pl.BlockSpec((TILE, F), lambda i, te_s: (i, 0)),
        ),
        out_shape=jax.ShapeDtypeStruct((PADT, F), jnp.float32),
        compiler_params=pltpu.CompilerParams(
            vmem_limit_bytes=100 * 1024 * 1024),
    )(te, xs, expert_w1, b1r)
    y = pl.pallas_call(
        _gmm2_body,
        grid_spec=pltpu.PrefetchScalarGridSpec(
            num_scalar_prefetch=1,
            grid=(NT,),
            in_specs=[pl.BlockSpec((TILE, F), lambda i, te_s: (i, 0)),
                      pl.BlockSpec((1, F, D), lambda i, te_s: (te_s[i], 0, 0)),
                      pl.BlockSpec((1, 1, D), lambda i, te_s: (te_s[i], 0, 0))],
            out_specs=pl.BlockSpec((TILE, D), lambda i, te_s: (i, 0)),
        ),
        out_shape=jax.ShapeDtypeStruct((PADT, D), jnp.float32),
        compiler_params=pltpu.CompilerParams(
            vmem_limit_bytes=100 * 1024 * 1024),
    )(te, h1, expert_w2, b2r)

    idx_all = jnp.concatenate([d0, d1], axis=0)
    g = _sc_combine(y, idx_all)

    out = pl.pallas_call(
        _fin_body,
        grid=(S // 128,),
        in_specs=[pl.BlockSpec((128, D), lambda i: (i, 0)),
                  pl.BlockSpec((128, D), lambda i: (i, 0)),
                  pl.BlockSpec((128, D), lambda i: (i + S // 128, 0)),
                  pl.BlockSpec((128, 128), lambda i: (i, 0))],
        out_specs=pl.BlockSpec((128, D), lambda i: (i, 0)),
        out_shape=jax.ShapeDtypeStruct((S, D), jnp.float32),
    )(hidden, g, g, cwA)
    return out.reshape(1, S, D)


# speedup vs baseline: 1.4163x; 1.0581x over previous
import functools

import jax
import jax.numpy as jnp
from jax import lax
from jax.experimental import pallas as pl
from jax.experimental.pallas import tpu as pltpu
from jax.experimental.pallas import tpu_sc as plsc

S = 2048
D = 1024
H = 16
HD = 64
E = 8
F = 4096
T = S
TILE = 128
NT = 40
PADT = NT * TILE
NEG = -1e30

_NC = 2
_NW = _NC * 16
_TPW = T // _NW


def _bdot(a, b):
    return jnp.dot(a.astype(jnp.bfloat16), b.astype(jnp.bfloat16),
                   preferred_element_type=jnp.float32)

def _qkv_body(x_ref, w_ref, b_ref, o_ref):
    o_ref[...] = (_bdot(x_ref[...], w_ref[...])
                  + b_ref[0:1, :]).astype(jnp.bfloat16)


def _make_attn_body(row0, kw):
    def body(q_ref, k_ref, v_ref, o_ref):
        qi = pl.program_id(1)
        q = q_ref[0]
        s = lax.dot_general(q, k_ref[0], (((1,), (1,)), ((), ())),
                            preferred_element_type=jnp.float32) * (1.0 / 8.0)
        row = row0 + qi * 256 + lax.broadcasted_iota(jnp.int32, (256, kw), 0)
        col = lax.broadcasted_iota(jnp.int32, (256, kw), 1)
        s = jnp.where(col <= row, s, NEG)
        m = jnp.max(s, axis=-1, keepdims=True)
        p = jnp.exp(s - m)
        l = jnp.sum(p, axis=-1, keepdims=True)
        o_ref[0] = (_bdot(p, v_ref[0]) * (1.0 / l)).astype(jnp.bfloat16)
    return body


def _attn_call(q, k, v, row0, kw):
    nq = (S - row0) // 256 if row0 else (S // 2) // 256
    return pl.pallas_call(
        _make_attn_body(row0, kw),
        grid=(H, nq),
        in_specs=[pl.BlockSpec((1, 256, HD), lambda h, i: (h, i + row0 // 256, 0)),
                  pl.BlockSpec((1, kw, HD), lambda h, i: (h, 0, 0)),
                  pl.BlockSpec((1, kw, HD), lambda h, i: (h, 0, 0))],
        out_specs=pl.BlockSpec((1, 256, HD), lambda h, i: (h, i, 0)),
        out_shape=jax.ShapeDtypeStruct((H, nq * 256, HD), jnp.bfloat16),
        compiler_params=pltpu.CompilerParams(
            vmem_limit_bytes=100 * 1024 * 1024),
    )(q, k, v)


def _proj_body(c_ref, w_ref, b_ref, x_ref, o_ref):
    o_ref[...] = x_ref[...] + b_ref[0:1, :] + _bdot(c_ref[...], w_ref[...])


def _route_body(h_ref, rw_ref, rb_ref, cw_ref, dest_ref, te_ref):
    hdn = h_ref[...]
    logits = _bdot(hdn, rw_ref[...]) + rb_ref[0:1, :]
    lane = lax.broadcasted_iota(jnp.int32, (T, 128), 1)
    logits = jnp.where(lane < E, logits, NEG)
    m = jnp.max(logits, axis=-1, keepdims=True)
    p = jnp.exp(logits - m)
    probs = p / jnp.sum(p, axis=-1, keepdims=True)
    m1 = jnp.max(probs, axis=-1, keepdims=True)
    i1 = jnp.min(jnp.where(probs == m1, lane, 128), axis=-1, keepdims=True)
    probs2 = jnp.where(lane == i1, -1.0, probs)
    m2 = jnp.max(probs2, axis=-1, keepdims=True)
    i2 = jnp.min(jnp.where(probs2 == m2, lane, 128), axis=-1, keepdims=True)
    den = m1 + m2
    cw1 = m1 / den
    cw2 = m2 / den
    am = ((lane == i1) | (lane == i2)).astype(jnp.int32)
    cs = am
    sh = 1
    while sh < T:
        z = jnp.zeros((sh, 128), jnp.int32)
        cs = cs + jnp.concatenate([z, cs[:-sh, :]], axis=0)
        sh *= 2
    ranks = cs - am
    counts = cs[T - 1:T, :]
    tiles_e = (counts + (TILE - 1)) // TILE
    tef = jnp.broadcast_to(tiles_e.astype(jnp.float32), (8, 128))
    r_i = lax.broadcasted_iota(jnp.int32, (128, 128), 0)
    c_i = lax.broadcasted_iota(jnp.int32, (128, 128), 1)
    tri = (r_i <= c_i).astype(jnp.float32)
    cum_i = jnp.dot(tef, tri,
                    preferred_element_type=jnp.float32)[0:1, :].astype(jnp.int32)
    po = (cum_i - tiles_e) * TILE
    slot = po + ranks
    d1 = jnp.sum(jnp.where(lane == i1, slot, 0), axis=-1, keepdims=True)
    d2 = jnp.sum(jnp.where(lane == i2, slot, 0), axis=-1, keepdims=True)
    cw_ref[...] = jnp.where(lane == 0, cw1, jnp.where(lane == 1, cw2, 0.0))
    dest_ref[...] = jnp.where(lane == 0, d1, jnp.where(lane == 1, d2, 0))
    eye = (lax.broadcasted_iota(jnp.int32, (8, 128), 0)
           == lax.broadcasted_iota(jnp.int32, (8, 128), 1))
    cum_col = jnp.sum(jnp.where(eye, jnp.broadcast_to(cum_i, (8, 128)), 0),
                      axis=-1, keepdims=True)
    lane8 = lax.broadcasted_iota(jnp.int32, (8, 128), 1)
    te2d = jnp.sum((lane8 >= cum_col).astype(jnp.int32), axis=0, keepdims=True)
    te_ref[...] = jnp.broadcast_to(jnp.minimum(te2d, E - 1), (8, 128))


def _gmm_body(te_sref, x_ref, w1_ref, b1_ref, w2_ref, b2_ref, o_ref):
    h1 = _bdot(x_ref[...], w1_ref[0]) + b1_ref[0]
    h1 = jax.nn.gelu(h1, approximate=True)
    o_ref[...] = _bdot(h1, w2_ref[0]) + b2_ref[0]


def _fin_body(h_ref, g0_ref, g1_ref, cw_ref, o_ref):
    o_ref[...] = (h_ref[...] + cw_ref[:, 0:1] * g0_ref[...]
                  + cw_ref[:, 1:2] * g1_ref[...])


def _sc_mesh():
    return plsc.VectorSubcoreMesh(core_axis_name="c", subcore_axis_name="s")


def _sc_dispatch(hidden, d0, d1):
    @functools.partial(
        pl.kernel, mesh=_sc_mesh(),
        out_type=jax.ShapeDtypeStruct((PADT, D), jnp.float32),
        scratch_types=[pltpu.VMEM((_TPW, D), jnp.float32),
                       pltpu.VMEM((_TPW,), jnp.int32),
                       pltpu.SemaphoreType.DMA],
    )
    def run(h_hbm, d0_hbm, d1_hbm, x_hbm, h_v, idx_v, sem):
        wid = lax.axis_index("s") * _NC + lax.axis_index("c")
        base = wid * _TPW
        pltpu.sync_copy(h_hbm.at[pl.ds(base, _TPW)], h_v)
        pltpu.sync_copy(d0_hbm.at[pl.ds(base, _TPW)], idx_v)
        pltpu.async_copy(h_v, x_hbm.at[idx_v], sem).wait()
        pltpu.sync_copy(d1_hbm.at[pl.ds(base, _TPW)], idx_v)
        pltpu.async_copy(h_v, x_hbm.at[idx_v], sem).wait()

    return run(hidden, d0, d1)


def _sc_combine(y, idx_all):
    @functools.partial(
        pl.kernel, mesh=_sc_mesh(),
        out_type=jax.ShapeDtypeStruct((2 * T, D), jnp.float32),
        scratch_types=[pltpu.VMEM((_TPW, D), jnp.float32),
                       pltpu.VMEM((_TPW,), jnp.int32),
                       pltpu.SemaphoreType.DMA],
    )
    def run(y_hbm, idx_hbm, g_hbm, rows_v, idx_v, sem):
        wid = lax.axis_index("s") * _NC + lax.axis_index("c")
        for cch in range(2):
            base = wid * (2 * _TPW) + cch * _TPW
            pltpu.sync_copy(idx_hbm.at[pl.ds(base, _TPW)], idx_v)
            pltpu.async_copy(y_hbm.at[idx_v], rows_v, sem).wait()
            pltpu.sync_copy(rows_v, g_hbm.at[pl.ds(base, _TPW)])

    return run(y, idx_all)


def kernel(hidden_states, c_attn_w, c_attn_b, c_proj_w, c_proj_b,
           router_w, router_b, expert_w1, expert_b1, expert_w2, expert_b2):
    x = hidden_states.reshape(S, D)

    b2d = jnp.broadcast_to(c_attn_b.reshape(1, 3 * D), (8, 3 * D))
    xb = x.astype(jnp.bfloat16)
    wqkvb = c_attn_w.astype(jnp.bfloat16)
    qkv = pl.pallas_call(
        _qkv_body,
        grid=(6,),
        in_specs=[pl.BlockSpec((S, D), lambda j: (0, 0)),
                  pl.BlockSpec((D, 512), lambda j: (0, j)),
                  pl.BlockSpec((8, 512), lambda j: (0, j))],
        out_specs=pl.BlockSpec((S, 512), lambda j: (0, j)),
        out_shape=jax.ShapeDtypeStruct((S, 3 * D), jnp.bfloat16),
        compiler_params=pltpu.CompilerParams(
            vmem_limit_bytes=100 * 1024 * 1024),
    )(xb, wqkvb, b2d)

    q = qkv[:, 0:D].reshape(S, H, HD).transpose(1, 0, 2)
    k = qkv[:, D:2 * D].reshape(S, H, HD).transpose(1, 0, 2)
    v = qkv[:, 2 * D:].reshape(S, H, HD).transpose(1, 0, 2)
    ctx_lo = _attn_call(q, k, v, 0, S // 2)
    ctx_hi = _attn_call(q, k, v, S // 2, S)
    ctx = jnp.concatenate([ctx_lo, ctx_hi], axis=1)

    ctx_flat = ctx.transpose(1, 0, 2).reshape(S, D)
    bp = jnp.broadcast_to(c_proj_b.reshape(1, D), (8, D))
    wpb = c_proj_w.astype(jnp.bfloat16)
    hidden = pl.pallas_call(
        _proj_body,
        grid=(2, 8),
        in_specs=[pl.BlockSpec((256, D), lambda j, i: (i, 0)),
                  pl.BlockSpec((D, 512), lambda j, i: (0, j)),
                  pl.BlockSpec((8, 512), lambda j, i: (0, j)),
                  pl.BlockSpec((256, 512), lambda j, i: (i, j))],
        out_specs=pl.BlockSpec((256, 512), lambda j, i: (i, j)),
        out_shape=jax.ShapeDtypeStruct((S, D), jnp.float32),
    )(ctx_flat, wpb, bp, x)

    rw = jnp.pad(router_w, ((0, 0), (0, 128 - E)))
    rb = jnp.pad(router_b.reshape(1, E), ((0, 7), (0, 128 - E)))
    cwA, destB, teC = pl.pallas_call(
        _route_body,
        grid=(1,),
        in_specs=[pl.BlockSpec((T, D), lambda i: (0, 0)),
                  pl.BlockSpec((D, 128), lambda i: (0, 0)),
                  pl.BlockSpec((8, 128), lambda i: (0, 0))],
        out_specs=[pl.BlockSpec((T, 128), lambda i: (0, 0)),
                   pl.BlockSpec((T, 128), lambda i: (0, 0)),
                   pl.BlockSpec((8, 128), lambda i: (0, 0))],
        out_shape=[jax.ShapeDtypeStruct((T, 128), jnp.float32),
                   jax.ShapeDtypeStruct((T, 128), jnp.int32),
                   jax.ShapeDtypeStruct((8, 128), jnp.int32)],
        compiler_params=pltpu.CompilerParams(
            vmem_limit_bytes=100 * 1024 * 1024),
    )(hidden, rw, rb)
    d0 = destB[:, 0]
    d1 = destB[:, 1]
    te = teC[0, :NT]

    xs = _sc_dispatch(hidden, d0, d1)

    b1r = expert_b1.reshape(E, 1, F)
    b2r = expert_b2.reshape(E, 1, D)
    ew1b = expert_w1.astype(jnp.bfloat16)
    ew2b = expert_w2.astype(jnp.bfloat16)
    y = pl.pallas_call(
        _gmm_body,
        grid_spec=pltpu.PrefetchScalarGridSpec(
            num_scalar_prefetch=1,
            grid=(NT,),
            in_specs=[pl.BlockSpec((TILE, D), lambda i, te_s: (i, 0)),
                      pl.BlockSpec((1, D, F), lambda i, te_s: (te_s[i], 0, 0)),
                      pl.BlockSpec((1, 1, F), lambda i, te_s: (te_s[i], 0, 0)),
                      pl.BlockSpec((1, F, D), lambda i, te_s: (te_s[i], 0, 0)),
                      pl.BlockSpec((1, 1, D), lambda i, te_s: (te_s[i], 0, 0))],
            out_specs=pl.BlockSpec((TILE, D), lambda i, te_s: (i, 0)),
        ),
        out_shape=jax.ShapeDtypeStruct((PADT, D), jnp.float32),
        compiler_params=pltpu.CompilerParams(
            vmem_limit_bytes=110 * 1024 * 1024),
    )(te, xs, ew1b, b1r, ew2b, b2r)

    idx_all = jnp.concatenate([d0, d1], axis=0)
    g = _sc_combine(y, idx_all)

    out = pl.pallas_call(
        _fin_body,
        grid=(S // 128,),
        in_specs=[pl.BlockSpec((128, D), lambda i: (i, 0)),
                  pl.BlockSpec((128, D), lambda i: (i, 0)),
                  pl.BlockSpec((128, D), lambda i: (i + S // 128, 0)),
                  pl.BlockSpec((128, 128), lambda i: (i, 0))],
        out_specs=pl.BlockSpec((128, D), lambda i: (i, 0)),
        out_shape=jax.ShapeDtypeStruct((S, D), jnp.float32),
    )(hidden, g, g, cwA)
    return out.reshape(1, S, D)


# speedup vs baseline: 1.5164x; 1.0706x over previous
import functools

import jax
import jax.numpy as jnp
from jax import lax
from jax.experimental import pallas as pl
from jax.experimental.pallas import tpu as pltpu
from jax.experimental.pallas import tpu_sc as plsc

S = 2048
D = 1024
H = 16
HD = 64
E = 8
F = 4096
T = S
TILE = 128
NT = 40
PADT = NT * TILE
NEG = -1e30

_NC = 2
_NW = _NC * 16
_TPW = T // _NW


def _bdot(a, b):
    return jnp.dot(a.astype(jnp.bfloat16), b.astype(jnp.bfloat16),
                   preferred_element_type=jnp.float32)

def _qkv_body(x_ref, w_ref, b_ref, o_ref):
    o_ref[...] = (_bdot(x_ref[...], w_ref[...])
                  + b_ref[0:1, :]).astype(jnp.bfloat16)


def _make_attn_body(row0, kw):
    def body(q_ref, k_ref, v_ref, o_ref):
        qi = pl.program_id(1)
        q = q_ref[0]
        s = lax.dot_general(q, k_ref[0], (((1,), (1,)), ((), ())),
                            preferred_element_type=jnp.float32) * (1.0 / 8.0)
        row = row0 + qi * 256 + lax.broadcasted_iota(jnp.int32, (256, kw), 0)
        col = lax.broadcasted_iota(jnp.int32, (256, kw), 1)
        s = jnp.where(col <= row, s, NEG)
        m = jnp.max(s, axis=-1, keepdims=True)
        p = jnp.exp(s - m)
        l = jnp.sum(p, axis=-1, keepdims=True)
        o_ref[0] = (_bdot(p, v_ref[0]) * (1.0 / l)).astype(jnp.bfloat16)
    return body


def _attn_call(q, k, v, row0, kw, nq):
    return pl.pallas_call(
        _make_attn_body(row0, kw),
        grid=(H, nq),
        in_specs=[pl.BlockSpec((1, 256, HD), lambda h, i: (h, i + row0 // 256, 0)),
                  pl.BlockSpec((1, kw, HD), lambda h, i: (h, 0, 0)),
                  pl.BlockSpec((1, kw, HD), lambda h, i: (h, 0, 0))],
        out_specs=pl.BlockSpec((1, 256, HD), lambda h, i: (h, i, 0)),
        out_shape=jax.ShapeDtypeStruct((H, nq * 256, HD), jnp.bfloat16),
        compiler_params=pltpu.CompilerParams(
            vmem_limit_bytes=100 * 1024 * 1024),
    )(q, k, v)


def _proj_body(c_ref, w_ref, b_ref, x_ref, o_ref):
    o_ref[...] = x_ref[...] + b_ref[0:1, :] + _bdot(c_ref[...], w_ref[...])


def _route_body(h_ref, rw_ref, rb_ref, cw_ref, dest_ref, te_ref):
    hdn = h_ref[...]
    logits = _bdot(hdn, rw_ref[...]) + rb_ref[0:1, :]
    lane = lax.broadcasted_iota(jnp.int32, (T, 128), 1)
    logits = jnp.where(lane < E, logits, NEG)
    m = jnp.max(logits, axis=-1, keepdims=True)
    p = jnp.exp(logits - m)
    probs = p / jnp.sum(p, axis=-1, keepdims=True)
    m1 = jnp.max(probs, axis=-1, keepdims=True)
    i1 = jnp.min(jnp.where(probs == m1, lane, 128), axis=-1, keepdims=True)
    probs2 = jnp.where(lane == i1, -1.0, probs)
    m2 = jnp.max(probs2, axis=-1, keepdims=True)
    i2 = jnp.min(jnp.where(probs2 == m2, lane, 128), axis=-1, keepdims=True)
    den = m1 + m2
    cw1 = m1 / den
    cw2 = m2 / den
    am = ((lane == i1) | (lane == i2)).astype(jnp.int32)
    cs = am
    sh = 1
    while sh < T:
        z = jnp.zeros((sh, 128), jnp.int32)
        cs = cs + jnp.concatenate([z, cs[:-sh, :]], axis=0)
        sh *= 2
    ranks = cs - am
    counts = cs[T - 1:T, :]
    tiles_e = (counts + (TILE - 1)) // TILE
    tef = jnp.broadcast_to(tiles_e.astype(jnp.float32), (8, 128))
    r_i = lax.broadcasted_iota(jnp.int32, (128, 128), 0)
    c_i = lax.broadcasted_iota(jnp.int32, (128, 128), 1)
    tri = (r_i <= c_i).astype(jnp.float32)
    cum_i = jnp.dot(tef, tri,
                    preferred_element_type=jnp.float32)[0:1, :].astype(jnp.int32)
    po = (cum_i - tiles_e) * TILE
    slot = po + ranks
    d1 = jnp.sum(jnp.where(lane == i1, slot, 0), axis=-1, keepdims=True)
    d2 = jnp.sum(jnp.where(lane == i2, slot, 0), axis=-1, keepdims=True)
    cw_ref[...] = jnp.where(lane == 0, cw1, jnp.where(lane == 1, cw2, 0.0))
    dest_ref[...] = jnp.where(lane == 0, d1, jnp.where(lane == 1, d2, 0))
    eye = (lax.broadcasted_iota(jnp.int32, (8, 128), 0)
           == lax.broadcasted_iota(jnp.int32, (8, 128), 1))
    cum_col = jnp.sum(jnp.where(eye, jnp.broadcast_to(cum_i, (8, 128)), 0),
                      axis=-1, keepdims=True)
    lane8 = lax.broadcasted_iota(jnp.int32, (8, 128), 1)
    te2d = jnp.sum((lane8 >= cum_col).astype(jnp.int32), axis=0, keepdims=True)
    te_ref[...] = jnp.broadcast_to(jnp.minimum(te2d, E - 1), (8, 128))


def _gmm1_body(te_sref, x_ref, w1_ref, b1_ref, o_ref):
    h1 = _bdot(x_ref[...], w1_ref[0]) + b1_ref[0]
    o_ref[...] = jax.nn.gelu(h1, approximate=True).astype(jnp.bfloat16)


def _gmm2_body(te_sref, h1_ref, w2_ref, b2_ref, o_ref):
    o_ref[...] = _bdot(h1_ref[...], w2_ref[0]) + b2_ref[0]


def _fin_body(h_ref, g0_ref, g1_ref, cw_ref, o_ref):
    o_ref[...] = (h_ref[...] + cw_ref[:, 0:1] * g0_ref[...]
                  + cw_ref[:, 1:2] * g1_ref[...])


def _sc_mesh():
    return plsc.VectorSubcoreMesh(core_axis_name="c", subcore_axis_name="s")


def _sc_dispatch(hidden, d0, d1):
    @functools.partial(
        pl.kernel, mesh=_sc_mesh(),
        out_type=jax.ShapeDtypeStruct((PADT, D), jnp.float32),
        scratch_types=[pltpu.VMEM((_TPW, D), jnp.float32),
                       pltpu.VMEM((_TPW,), jnp.int32),
                       pltpu.SemaphoreType.DMA],
    )
    def run(h_hbm, d0_hbm, d1_hbm, x_hbm, h_v, idx_v, sem):
        wid = lax.axis_index("s") * _NC + lax.axis_index("c")
        base = wid * _TPW
        pltpu.sync_copy(h_hbm.at[pl.ds(base, _TPW)], h_v)
        pltpu.sync_copy(d0_hbm.at[pl.ds(base, _TPW)], idx_v)
        pltpu.async_copy(h_v, x_hbm.at[idx_v], sem).wait()
        pltpu.sync_copy(d1_hbm.at[pl.ds(base, _TPW)], idx_v)
        pltpu.async_copy(h_v, x_hbm.at[idx_v], sem).wait()

    return run(hidden, d0, d1)


def _sc_combine(y, idx_all):
    @functools.partial(
        pl.kernel, mesh=_sc_mesh(),
        out_type=jax.ShapeDtypeStruct((2 * T, D), jnp.float32),
        scratch_types=[pltpu.VMEM((_TPW, D), jnp.float32),
                       pltpu.VMEM((_TPW,), jnp.int32),
                       pltpu.SemaphoreType.DMA],
    )
    def run(y_hbm, idx_hbm, g_hbm, rows_v, idx_v, sem):
        wid = lax.axis_index("s") * _NC + lax.axis_index("c")
        for cch in range(2):
            base = wid * (2 * _TPW) + cch * _TPW
            pltpu.sync_copy(idx_hbm.at[pl.ds(base, _TPW)], idx_v)
            pltpu.async_copy(y_hbm.at[idx_v], rows_v, sem).wait()
            pltpu.sync_copy(rows_v, g_hbm.at[pl.ds(base, _TPW)])

    return run(y, idx_all)


def kernel(hidden_states, c_attn_w, c_attn_b, c_proj_w, c_proj_b,
           router_w, router_b, expert_w1, expert_b1, expert_w2, expert_b2):
    x = hidden_states.reshape(S, D)

    b2d = jnp.broadcast_to(c_attn_b.reshape(1, 3 * D), (8, 3 * D))
    xb = x.astype(jnp.bfloat16)
    wqkvb = c_attn_w.astype(jnp.bfloat16)
    qkv = pl.pallas_call(
        _qkv_body,
        grid=(6,),
        in_specs=[pl.BlockSpec((S, D), lambda j: (0, 0)),
                  pl.BlockSpec((D, 512), lambda j: (0, j)),
                  pl.BlockSpec((8, 512), lambda j: (0, j))],
        out_specs=pl.BlockSpec((S, 512), lambda j: (0, j)),
        out_shape=jax.ShapeDtypeStruct((S, 3 * D), jnp.bfloat16),
        compiler_params=pltpu.CompilerParams(
            vmem_limit_bytes=100 * 1024 * 1024),
    )(xb, wqkvb, b2d)

    q = qkv[:, 0:D].reshape(S, H, HD).transpose(1, 0, 2)
    k = qkv[:, D:2 * D].reshape(S, H, HD).transpose(1, 0, 2)
    v = qkv[:, 2 * D:].reshape(S, H, HD).transpose(1, 0, 2)
    k15 = k[:, :1536]
    v15 = v[:, :1536]
    ctx = jnp.concatenate([
        _attn_call(q, k, v, 0, 512, 2),
        _attn_call(q, k, v, 512, 1024, 2),
        _attn_call(q, k15, v15, 1024, 1536, 2),
        _attn_call(q, k, v, 1536, 2048, 2),
    ], axis=1)

    ctx_flat = ctx.transpose(1, 0, 2).reshape(S, D)
    bp = jnp.broadcast_to(c_proj_b.reshape(1, D), (8, D))
    wpb = c_proj_w.astype(jnp.bfloat16)
    hidden = pl.pallas_call(
        _proj_body,
        grid=(2, 8),
        in_specs=[pl.BlockSpec((256, D), lambda j, i: (i, 0)),
                  pl.BlockSpec((D, 512), lambda j, i: (0, j)),
                  pl.BlockSpec((8, 512), lambda j, i: (0, j)),
                  pl.BlockSpec((256, 512), lambda j, i: (i, j))],
        out_specs=pl.BlockSpec((256, 512), lambda j, i: (i, j)),
        out_shape=jax.ShapeDtypeStruct((S, D), jnp.float32),
    )(ctx_flat, wpb, bp, x)

    rw = jnp.pad(router_w, ((0, 0), (0, 128 - E)))
    rb = jnp.pad(router_b.reshape(1, E), ((0, 7), (0, 128 - E)))
    cwA, destB, teC = pl.pallas_call(
        _route_body,
        grid=(1,),
        in_specs=[pl.BlockSpec((T, D), lambda i: (0, 0)),
                  pl.BlockSpec((D, 128), lambda i: (0, 0)),
                  pl.BlockSpec((8, 128), lambda i: (0, 0))],
        out_specs=[pl.BlockSpec((T, 128), lambda i: (0, 0)),
                   pl.BlockSpec((T, 128), lambda i: (0, 0)),
                   pl.BlockSpec((8, 128), lambda i: (0, 0))],
        out_shape=[jax.ShapeDtypeStruct((T, 128), jnp.float32),
                   jax.ShapeDtypeStruct((T, 128), jnp.int32),
                   jax.ShapeDtypeStruct((8, 128), jnp.int32)],
        compiler_params=pltpu.CompilerParams(
            vmem_limit_bytes=100 * 1024 * 1024),
    )(hidden, rw, rb)
    d0 = destB[:, 0]
    d1 = destB[:, 1]
    te = teC[0, :NT]

    xs = _sc_dispatch(hidden, d0, d1)

    b1r = expert_b1.reshape(E, 1, F)
    b2r = expert_b2.reshape(E, 1, D)
    h1 = pl.pallas_call(
        _gmm1_body,
        grid_spec=pltpu.PrefetchScalarGridSpec(
            num_scalar_prefetch=1,
            grid=(NT,),
            in_specs=[pl.BlockSpec((TILE, D), lambda i, te_s: (i, 0)),
                      pl.BlockSpec((1, D, F), lambda i, te_s: (te_s[i], 0, 0)),
                      pl.BlockSpec((1, 1, F), lambda i, te_s: (te_s[i], 0, 0))],
            out_specs=pl.BlockSpec((TILE, F), lambda i, te_s: (i, 0)),
        ),
        out_shape=jax.ShapeDtypeStruct((PADT, F), jnp.bfloat16),
        compiler_params=pltpu.CompilerParams(
            vmem_limit_bytes=110 * 1024 * 1024),
    )(te, xs, expert_w1, b1r)
    y = pl.pallas_call(
        _gmm2_body,
        grid_spec=pltpu.PrefetchScalarGridSpec(
            num_scalar_prefetch=1,
            grid=(NT,),
            in_specs=[pl.BlockSpec((TILE, F), lambda i, te_s: (i, 0)),
                      pl.BlockSpec((1, F, D), lambda i, te_s: (te_s[i], 0, 0)),
                      pl.BlockSpec((1, 1, D), lambda i, te_s: (te_s[i], 0, 0))],
            out_specs=pl.BlockSpec((TILE, D), lambda i, te_s: (i, 0)),
        ),
        out_shape=jax.ShapeDtypeStruct((PADT, D), jnp.float32),
        compiler_params=pltpu.CompilerParams(
            vmem_limit_bytes=110 * 1024 * 1024),
    )(te, h1, expert_w2, b2r)

    idx_all = jnp.concatenate([d0, d1], axis=0)
    g = _sc_combine(y, idx_all)

    out = pl.pallas_call(
        _fin_body,
        grid=(S // 128,),
        in_specs=[pl.BlockSpec((128, D), lambda i: (i, 0)),
                  pl.BlockSpec((128, D), lambda i: (i, 0)),
                  pl.BlockSpec((128, D), lambda i: (i + S // 128, 0)),
                  pl.BlockSpec((128, 128), lambda i: (i, 0))],
        out_specs=pl.BlockSpec((128, D), lambda i: (i, 0)),
        out_shape=jax.ShapeDtypeStruct((S, D), jnp.float32),
    )(hidden, g, g, cwA)
    return out.reshape(1, S, D)


# speedup vs baseline: 1.7404x; 1.1477x over previous
import functools

import jax
import jax.numpy as jnp
from jax import lax
from jax.experimental import pallas as pl
from jax.experimental.pallas import tpu as pltpu
from jax.experimental.pallas import tpu_sc as plsc

S = 2048
D = 1024
H = 16
HD = 64
E = 8
F = 4096
T = S
TILE = 128
NT = 40
PADT = NT * TILE
NEG = -1e30

_NC = 2
_NW = _NC * 16
_TPW = T // _NW


def _bdot(a, b):
    return jnp.dot(a.astype(jnp.bfloat16), b.astype(jnp.bfloat16),
                   preferred_element_type=jnp.float32)

def _qkv_body(x_ref, w_ref, b_ref, o_ref):
    o_ref[...] = (_bdot(x_ref[...], w_ref[...])
                  + b_ref[0:1, :]).astype(jnp.bfloat16)


def _make_attn_body(row0, kw):
    def body(q_ref, k_ref, v_ref, o_ref):
        qi = pl.program_id(1)
        q2 = q_ref[...]
        k2 = k_ref[...]
        v2 = v_ref[...]
        row = row0 + qi * 256 + lax.broadcasted_iota(jnp.int32, (256, kw), 0)
        col = lax.broadcasted_iota(jnp.int32, (256, kw), 1)
        causal = col <= row
        for sub in range(2):
            qh = q2[:, sub * HD:(sub + 1) * HD]
            kh = k2[:, sub * HD:(sub + 1) * HD]
            vh = v2[:, sub * HD:(sub + 1) * HD]
            s = lax.dot_general(qh, kh, (((1,), (1,)), ((), ())),
                                preferred_element_type=jnp.float32) * (1.0 / 8.0)
            s = jnp.where(causal, s, NEG)
            m = jnp.max(s, axis=-1, keepdims=True)
            p = jnp.exp(s - m)
            l = jnp.sum(p, axis=-1, keepdims=True)
            o_ref[:, sub * HD:(sub + 1) * HD] = (
                _bdot(p, vh) * (1.0 / l)).astype(jnp.bfloat16)
    return body


def _attn_call(qkv, row0, kw, nq):
    r0b = row0 // 256
    return pl.pallas_call(
        _make_attn_body(row0, kw),
        grid=(H // 2, nq),
        in_specs=[pl.BlockSpec((256, 128), lambda h, i: (i + r0b, h)),
                  pl.BlockSpec((kw, 128), lambda h, i: (0, 8 + h)),
                  pl.BlockSpec((kw, 128), lambda h, i: (0, 16 + h))],
        out_specs=pl.BlockSpec((256, 128), lambda h, i: (i, h)),
        out_shape=jax.ShapeDtypeStruct((nq * 256, D), jnp.bfloat16),
        compiler_params=pltpu.CompilerParams(
            vmem_limit_bytes=100 * 1024 * 1024),
    )(qkv, qkv, qkv)


def _proj_body(c_ref, w_ref, b_ref, x_ref, o_ref):
    o_ref[...] = x_ref[...] + b_ref[0:1, :] + _bdot(c_ref[...], w_ref[...])


def _route_body(h_ref, rw_ref, rb_ref, cw_ref, dest_ref, te_ref):
    hdn = h_ref[...]
    logits = _bdot(hdn, rw_ref[...]) + rb_ref[0:1, :]
    lane = lax.broadcasted_iota(jnp.int32, (T, 128), 1)
    logits = jnp.where(lane < E, logits, NEG)
    m = jnp.max(logits, axis=-1, keepdims=True)
    p = jnp.exp(logits - m)
    probs = p / jnp.sum(p, axis=-1, keepdims=True)
    m1 = jnp.max(probs, axis=-1, keepdims=True)
    i1 = jnp.min(jnp.where(probs == m1, lane, 128), axis=-1, keepdims=True)
    probs2 = jnp.where(lane == i1, -1.0, probs)
    m2 = jnp.max(probs2, axis=-1, keepdims=True)
    i2 = jnp.min(jnp.where(probs2 == m2, lane, 128), axis=-1, keepdims=True)
    den = m1 + m2
    cw1 = m1 / den
    cw2 = m2 / den
    am = ((lane == i1) | (lane == i2)).astype(jnp.int32)
    cs = am
    sh = 1
    while sh < T:
        z = jnp.zeros((sh, 128), jnp.int32)
        cs = cs + jnp.concatenate([z, cs[:-sh, :]], axis=0)
        sh *= 2
    ranks = cs - am
    counts = cs[T - 1:T, :]
    tiles_e = (counts + (TILE - 1)) // TILE
    tef = jnp.broadcast_to(tiles_e.astype(jnp.float32), (8, 128))
    r_i = lax.broadcasted_iota(jnp.int32, (128, 128), 0)
    c_i = lax.broadcasted_iota(jnp.int32, (128, 128), 1)
    tri = (r_i <= c_i).astype(jnp.float32)
    cum_i = jnp.dot(tef, tri,
                    preferred_element_type=jnp.float32)[0:1, :].astype(jnp.int32)
    po = (cum_i - tiles_e) * TILE
    slot = po + ranks
    d1 = jnp.sum(jnp.where(lane == i1, slot, 0), axis=-1, keepdims=True)
    d2 = jnp.sum(jnp.where(lane == i2, slot, 0), axis=-1, keepdims=True)
    cw_ref[...] = jnp.where(lane == 0, cw1, jnp.where(lane == 1, cw2, 0.0))
    dest_ref[...] = jnp.where(lane == 0, d1, jnp.where(lane == 1, d2, 0))
    eye = (lax.broadcasted_iota(jnp.int32, (8, 128), 0)
           == lax.broadcasted_iota(jnp.int32, (8, 128), 1))
    cum_col = jnp.sum(jnp.where(eye, jnp.broadcast_to(cum_i, (8, 128)), 0),
                      axis=-1, keepdims=True)
    lane8 = lax.broadcasted_iota(jnp.int32, (8, 128), 1)
    te2d = jnp.sum((lane8 >= cum_col).astype(jnp.int32), axis=0, keepdims=True)
    te_ref[...] = jnp.broadcast_to(jnp.minimum(te2d, E - 1), (8, 128))


def _gmm1_body(te_sref, x_ref, w1_ref, b1_ref, o_ref, w1b_ref):
    i = pl.program_id(0)
    changed = (i == 0) | (te_sref[i] != te_sref[jnp.maximum(i - 1, 0)])

    @pl.when(changed)
    def _():
        w1b_ref[...] = w1_ref[0].astype(jnp.bfloat16)

    h1 = jnp.dot(x_ref[...].astype(jnp.bfloat16), w1b_ref[...],
                 preferred_element_type=jnp.float32) + b1_ref[0]
    o_ref[...] = jax.nn.gelu(h1, approximate=True).astype(jnp.bfloat16)


def _gmm2_body(te_sref, h1_ref, w2_ref, b2_ref, o_ref, w2b_ref):
    i = pl.program_id(0)
    changed = (i == 0) | (te_sref[i] != te_sref[jnp.maximum(i - 1, 0)])

    @pl.when(changed)
    def _():
        w2b_ref[...] = w2_ref[0].astype(jnp.bfloat16)

    o_ref[...] = jnp.dot(h1_ref[...], w2b_ref[...],
                         preferred_element_type=jnp.float32) + b2_ref[0]


def _fin_body(h_ref, g0_ref, g1_ref, cw_ref, o_ref):
    o_ref[...] = (h_ref[...] + cw_ref[:, 0:1] * g0_ref[...]
                  + cw_ref[:, 1:2] * g1_ref[...])


def _sc_mesh():
    return plsc.VectorSubcoreMesh(core_axis_name="c", subcore_axis_name="s")


def _sc_dispatch(hidden, d0, d1):
    @functools.partial(
        pl.kernel, mesh=_sc_mesh(),
        out_type=jax.ShapeDtypeStruct((PADT, D), jnp.float32),
        scratch_types=[pltpu.VMEM((_TPW, D), jnp.float32),
                       pltpu.VMEM((_TPW,), jnp.int32),
                       pltpu.VMEM((_TPW,), jnp.int32),
                       pltpu.SemaphoreType.DMA,
                       pltpu.SemaphoreType.DMA],
    )
    def run(h_hbm, d0_hbm, d1_hbm, x_hbm, h_v, i0_v, i1_v, s0, s1):
        wid = lax.axis_index("s") * _NC + lax.axis_index("c")
        base = wid * _TPW
        pltpu.sync_copy(h_hbm.at[pl.ds(base, _TPW)], h_v)
        pltpu.sync_copy(d0_hbm.at[pl.ds(base, _TPW)], i0_v)
        pltpu.sync_copy(d1_hbm.at[pl.ds(base, _TPW)], i1_v)
        c0 = pltpu.async_copy(h_v, x_hbm.at[i0_v], s0)
        c1 = pltpu.async_copy(h_v, x_hbm.at[i1_v], s1)
        c0.wait()
        c1.wait()

    return run(hidden, d0, d1)


def _sc_combine(y, idx_all):
    @functools.partial(
        pl.kernel, mesh=_sc_mesh(),
        out_type=jax.ShapeDtypeStruct((2 * T, D), jnp.float32),
        scratch_types=[pltpu.VMEM((_TPW, D), jnp.float32),
                       pltpu.VMEM((_TPW,), jnp.int32),
                       pltpu.SemaphoreType.DMA],
    )
    def run(y_hbm, idx_hbm, g_hbm, rows_v, idx_v, sem):
        wid = lax.axis_index("s") * _NC + lax.axis_index("c")
        for cch in range(2):
            base = wid * (2 * _TPW) + cch * _TPW
            pltpu.sync_copy(idx_hbm.at[pl.ds(base, _TPW)], idx_v)
            pltpu.async_copy(y_hbm.at[idx_v], rows_v, sem).wait()
            pltpu.sync_copy(rows_v, g_hbm.at[pl.ds(base, _TPW)])

    return run(y, idx_all)


def kernel(hidden_states, c_attn_w, c_attn_b, c_proj_w, c_proj_b,
           router_w, router_b, expert_w1, expert_b1, expert_w2, expert_b2):
    x = hidden_states.reshape(S, D)

    b2d = jnp.broadcast_to(c_attn_b.reshape(1, 3 * D), (8, 3 * D))
    xb = x.astype(jnp.bfloat16)
    wqkvb = c_attn_w.astype(jnp.bfloat16)
    qkv = pl.pallas_call(
        _qkv_body,
        grid=(6,),
        in_specs=[pl.BlockSpec((S, D), lambda j: (0, 0)),
                  pl.BlockSpec((D, 512), lambda j: (0, j)),
                  pl.BlockSpec((8, 512), lambda j: (0, j))],
        out_specs=pl.BlockSpec((S, 512), lambda j: (0, j)),
        out_shape=jax.ShapeDtypeStruct((S, 3 * D), jnp.bfloat16),
        compiler_params=pltpu.CompilerParams(
            vmem_limit_bytes=100 * 1024 * 1024),
    )(xb, wqkvb, b2d)

    ctx_flat = jnp.concatenate([
        _attn_call(qkv, 0, 1024, 4),
        _attn_call(qkv, 1024, 2048, 4),
    ], axis=0)
    bp = jnp.broadcast_to(c_proj_b.reshape(1, D), (8, D))
    wpb = c_proj_w.astype(jnp.bfloat16)
    hidden = pl.pallas_call(
        _proj_body,
        grid=(2, 8),
        in_specs=[pl.BlockSpec((256, D), lambda j, i: (i, 0)),
                  pl.BlockSpec((D, 512), lambda j, i: (0, j)),
                  pl.BlockSpec((8, 512), lambda j, i: (0, j)),
                  pl.BlockSpec((256, 512), lambda j, i: (i, j))],
        out_specs=pl.BlockSpec((256, 512), lambda j, i: (i, j)),
        out_shape=jax.ShapeDtypeStruct((S, D), jnp.float32),
    )(ctx_flat, wpb, bp, x)

    rw = jnp.pad(router_w, ((0, 0), (0, 128 - E)))
    rb = jnp.pad(router_b.reshape(1, E), ((0, 7), (0, 128 - E)))
    cwA, destB, teC = pl.pallas_call(
        _route_body,
        grid=(1,),
        in_specs=[pl.BlockSpec((T, D), lambda i: (0, 0)),
                  pl.BlockSpec((D, 128), lambda i: (0, 0)),
                  pl.BlockSpec((8, 128), lambda i: (0, 0))],
        out_specs=[pl.BlockSpec((T, 128), lambda i: (0, 0)),
                   pl.BlockSpec((T, 128), lambda i: (0, 0)),
                   pl.BlockSpec((8, 128), lambda i: (0, 0))],
        out_shape=[jax.ShapeDtypeStruct((T, 128), jnp.float32),
                   jax.ShapeDtypeStruct((T, 128), jnp.int32),
                   jax.ShapeDtypeStruct((8, 128), jnp.int32)],
        compiler_params=pltpu.CompilerParams(
            vmem_limit_bytes=100 * 1024 * 1024),
    )(hidden, rw, rb)
    d0 = destB[:, 0]
    d1 = destB[:, 1]
    te = teC[0, :NT]

    xs = _sc_dispatch(hidden, d0, d1)

    b1r = expert_b1.reshape(E, 1, F)
    b2r = expert_b2.reshape(E, 1, D)
    h1 = pl.pallas_call(
        _gmm1_body,
        grid_spec=pltpu.PrefetchScalarGridSpec(
            num_scalar_prefetch=1,
            grid=(NT,),
            in_specs=[pl.BlockSpec((TILE, D), lambda i, te_s: (i, 0)),
                      pl.BlockSpec((1, D, F), lambda i, te_s: (te_s[i], 0, 0)),
                      pl.BlockSpec((1, 1, F), lambda i, te_s: (te_s[i], 0, 0))],
            out_specs=pl.BlockSpec((TILE, F), lambda i, te_s: (i, 0)),
            scratch_shapes=[pltpu.VMEM((D, F), jnp.bfloat16)],
        ),
        out_shape=jax.ShapeDtypeStruct((PADT, F), jnp.bfloat16),
        compiler_params=pltpu.CompilerParams(
            vmem_limit_bytes=110 * 1024 * 1024),
    )(te, xs, expert_w1, b1r)
    y = pl.pallas_call(
        _gmm2_body,
        grid_spec=pltpu.PrefetchScalarGridSpec(
            num_scalar_prefetch=1,
            grid=(NT,),
            in_specs=[pl.BlockSpec((TILE, F), lambda i, te_s: (i, 0)),
                      pl.BlockSpec((1, F, D), lambda i, te_s: (te_s[i], 0, 0)),
                      pl.BlockSpec((1, 1, D), lambda i, te_s: (te_s[i], 0, 0))],
            out_specs=pl.BlockSpec((TILE, D), lambda i, te_s: (i, 0)),
            scratch_shapes=[pltpu.VMEM((F, D), jnp.bfloat16)],
        ),
        out_shape=jax.ShapeDtypeStruct((PADT, D), jnp.float32),
        compiler_params=pltpu.CompilerParams(
            vmem_limit_bytes=110 * 1024 * 1024),
    )(te, h1, expert_w2, b2r)

    idx_all = jnp.concatenate([d0, d1], axis=0)
    g = _sc_combine(y, idx_all)

    out = pl.pallas_call(
        _fin_body,
        grid=(S // 128,),
        in_specs=[pl.BlockSpec((128, D), lambda i: (i, 0)),
                  pl.BlockSpec((128, D), lambda i: (i, 0)),
                  pl.BlockSpec((128, D), lambda i: (i + S // 128, 0)),
                  pl.BlockSpec((128, 128), lambda i: (i, 0))],
        out_specs=pl.BlockSpec((128, D), lambda i: (i, 0)),
        out_shape=jax.ShapeDtypeStruct((S, D), jnp.float32),
    )(hidden, g, g, cwA)
    return out.reshape(1, S, D)
